# baseline (speedup 1.0000x reference)
import functools

import jax
import jax.numpy as jnp
from jax import lax
from jax.experimental import pallas as pl
from jax.experimental.pallas import tpu as pltpu
from jax.experimental.pallas import tpu_sc as plsc

_EPS = 1e-12


def _sc_gather(ids_flat, word_table):
    N = ids_flat.shape[0]
    H = word_table.shape[1]
    info = plsc.get_sparse_core_info()
    NC, NS = info.num_cores, info.num_subcores
    NW = NC * NS
    per_w = N // NW
    CH = next(c for c in range(128, 7, -8) if per_w % c == 0)
    n_ch = per_w // CH
    NB = next(b for b in (5, 4, 3, 2, 1) if n_ch % b == 0)

    mesh = plsc.VectorSubcoreMesh(core_axis_name="c", subcore_axis_name="s")

    @functools.partial(
        pl.kernel,
        mesh=mesh,
        out_type=jax.ShapeDtypeStruct((N, H), jnp.float32),
        scratch_types=[
            pltpu.VMEM((NB, CH), jnp.int32),
            pltpu.VMEM((NB, CH, H), jnp.float32),
        ]
        + [pltpu.SemaphoreType.DMA] * (2 * NB),
    )
    def k(ids_hbm, table_hbm, out_hbm, idx_v, rows_v, *sems):
        wid = lax.axis_index("s") * NC + lax.axis_index("c")
        base = wid * per_w
        gsem = sems[:NB]
        wsem = sems[NB:]

        for b in range(NB):
            off = base + b * CH
            pltpu.sync_copy(ids_hbm.at[pl.ds(off, CH)], idx_v.at[b])
            pltpu.async_copy(table_hbm.at[idx_v.at[b]], rows_v.at[b], gsem[b])

        def body(i, carry):
            for b in range(NB):
                g = i * NB + b
                off = base + g * CH
                pltpu.make_async_copy(
                    table_hbm.at[idx_v.at[b]], rows_v.at[b], gsem[b]
                ).wait()
                pltpu.async_copy(
                    rows_v.at[b], out_hbm.at[pl.ds(off, CH)], wsem[b]
                )

                @pl.when(i < n_ch // NB - 1)
                def _prefetch():
                    noff = off + NB * CH
                    pltpu.sync_copy(ids_hbm.at[pl.ds(noff, CH)], idx_v.at[b])
                    pltpu.make_async_copy(
                        rows_v.at[b], out_hbm.at[pl.ds(off, CH)], wsem[b]
                    ).wait()
                    pltpu.async_copy(
                        table_hbm.at[idx_v.at[b]], rows_v.at[b], gsem[b]
                    )

            return carry

        lax.fori_loop(0, n_ch // NB, body, 0)
        for b in range(NB):
            off = base + (n_ch - NB + b) * CH
            pltpu.make_async_copy(
                rows_v.at[b], out_hbm.at[pl.ds(off, CH)], wsem[b]
            ).wait()

    return k(ids_flat, word_table)


def _tc_add_layernorm(gathered, pre_tab, pos_tab, gamma, beta, acc, boff, full_b):
    BK, L, H = gathered.shape
    RB = 64
    grid = (BK // RB,)
    blk = boff // RB

    def body(g_ref, pa_ref, pb_ref, gm_ref, bt_ref, *rest):
        o_ref = rest[-1]
        x = g_ref[...] + (pa_ref[...] + pb_ref[...])[None, :, :]
        u = jnp.mean(x, axis=-1, keepdims=True)
        s2 = jnp.mean((x - u) ** 2, axis=-1, keepdims=True)
        xn = (x - u) * lax.rsqrt(s2 + _EPS)
        o_ref[...] = xn * gm_ref[0][None, None, :] + bt_ref[0][None, None, :]

    in_specs = [
        pl.BlockSpec((RB, L, H), lambda i: (i, 0, 0)),
        pl.BlockSpec((L, H), lambda i: (0, 0)),
        pl.BlockSpec((L, H), lambda i: (0, 0)),
        pl.BlockSpec((1, H), lambda i: (0, 0)),
        pl.BlockSpec((1, H), lambda i: (0, 0)),
    ]
    args = [gathered, pre_tab, pos_tab, gamma, beta]
    aliases = {}
    if acc is not None:
        in_specs.append(pl.BlockSpec((RB, L, H), lambda i: (i + blk, 0, 0)))
        args.append(acc)
        aliases = {5: 0}

    return pl.pallas_call(
        body,
        grid=grid,
        in_specs=in_specs,
        out_specs=pl.BlockSpec((RB, L, H), lambda i: (i + blk, 0, 0)),
        out_shape=jax.ShapeDtypeStruct((full_b, L, H), jnp.float32),
        input_output_aliases=aliases,
    )(*args)


def kernel(input_ids, word_table, pretrained_table, pos_table, gamma, beta):
    B, L = input_ids.shape
    H = word_table.shape[1]
    chunks = (512, 384, 128)
    ids_flat = input_ids.reshape(-1).astype(jnp.int32)
    pre = pretrained_table[:L]
    pos = pos_table[:L]
    gm = gamma.reshape(1, H)
    bt = beta.reshape(1, H)

    offs = [0]
    for c in chunks:
        offs.append(offs[-1] + c)
    gathered = [
        _sc_gather(ids_flat[offs[k] * L:offs[k + 1] * L], word_table)
        for k in range(len(chunks))
    ]
    acc = None
    for k, BK in enumerate(chunks):
        acc = _tc_add_layernorm(
            gathered[k].reshape(BK, L, H), pre, pos, gm, bt, acc, offs[k], B
        )
    return acc


# speedup vs baseline: 1.0365x; 1.0365x over previous
import functools

import jax
import jax.numpy as jnp
from jax import lax
from jax.experimental import pallas as pl
from jax.experimental.pallas import tpu as pltpu
from jax.experimental.pallas import tpu_sc as plsc

_EPS = 1e-12


def _sc_gather(ids_flat, word_table):
    N = ids_flat.shape[0]
    H = word_table.shape[1]
    info = plsc.get_sparse_core_info()
    NC, NS = info.num_cores, info.num_subcores
    NW = NC * NS
    per_w = N // NW
    CH = next(c for c in range(128, 7, -8) if per_w % c == 0)
    n_ch = per_w // CH
    NB = next(b for b in (5, 4, 3, 2, 1) if n_ch % b == 0)

    mesh = plsc.VectorSubcoreMesh(core_axis_name="c", subcore_axis_name="s")

    @functools.partial(
        pl.kernel,
        mesh=mesh,
        out_type=jax.ShapeDtypeStruct((N, H), jnp.float32),
        scratch_types=[
            pltpu.VMEM((NB, CH), jnp.int32),
            pltpu.VMEM((NB, CH, H), jnp.float32),
        ]
        + [pltpu.SemaphoreType.DMA] * (2 * NB),
    )
    def k(ids_hbm, table_hbm, out_hbm, idx_v, rows_v, *sems):
        wid = lax.axis_index("s") * NC + lax.axis_index("c")
        base = wid * per_w
        gsem = sems[:NB]
        wsem = sems[NB:]

        for b in range(NB):
            off = base + b * CH
            pltpu.sync_copy(ids_hbm.at[pl.ds(off, CH)], idx_v.at[b])
            pltpu.async_copy(table_hbm.at[idx_v.at[b]], rows_v.at[b], gsem[b])

        def body(i, carry):
            for b in range(NB):
                g = i * NB + b
                off = base + g * CH
                pltpu.make_async_copy(
                    table_hbm.at[idx_v.at[b]], rows_v.at[b], gsem[b]
                ).wait()
                pltpu.async_copy(
                    rows_v.at[b], out_hbm.at[pl.ds(off, CH)], wsem[b]
                )

                @pl.when(i < n_ch // NB - 1)
                def _prefetch():
                    noff = off + NB * CH
                    pltpu.sync_copy(ids_hbm.at[pl.ds(noff, CH)], idx_v.at[b])
                    pltpu.make_async_copy(
                        rows_v.at[b], out_hbm.at[pl.ds(off, CH)], wsem[b]
                    ).wait()
                    pltpu.async_copy(
                        table_hbm.at[idx_v.at[b]], rows_v.at[b], gsem[b]
                    )

            return carry

        lax.fori_loop(0, n_ch // NB, body, 0)
        for b in range(NB):
            off = base + (n_ch - NB + b) * CH
            pltpu.make_async_copy(
                rows_v.at[b], out_hbm.at[pl.ds(off, CH)], wsem[b]
            ).wait()

    return k(ids_flat, word_table)


def _tc_add_layernorm(gathered, pre_tab, pos_tab, gamma, beta, acc, boff, full_b):
    BK, L, H = gathered.shape
    RB = 64
    grid = (BK // RB,)
    blk = boff // RB

    def body(g_ref, pa_ref, pb_ref, gm_ref, bt_ref, *rest):
        o_ref = rest[-1]
        x = g_ref[...] + (pa_ref[...] + pb_ref[...])[None, :, :]
        u = jnp.mean(x, axis=-1, keepdims=True)
        s2 = jnp.mean((x - u) ** 2, axis=-1, keepdims=True)
        xn = (x - u) * lax.rsqrt(s2 + _EPS)
        o_ref[...] = xn * gm_ref[0][None, None, :] + bt_ref[0][None, None, :]

    in_specs = [
        pl.BlockSpec((RB, L, H), lambda i: (i, 0, 0)),
        pl.BlockSpec((L, H), lambda i: (0, 0)),
        pl.BlockSpec((L, H), lambda i: (0, 0)),
        pl.BlockSpec((1, H), lambda i: (0, 0)),
        pl.BlockSpec((1, H), lambda i: (0, 0)),
    ]
    args = [gathered, pre_tab, pos_tab, gamma, beta]
    aliases = {}
    if acc is not None:
        in_specs.append(pl.BlockSpec((RB, L, H), lambda i: (i + blk, 0, 0)))
        args.append(acc)
        aliases = {5: 0}

    return pl.pallas_call(
        body,
        grid=grid,
        in_specs=in_specs,
        out_specs=pl.BlockSpec((RB, L, H), lambda i: (i + blk, 0, 0)),
        out_shape=jax.ShapeDtypeStruct((full_b, L, H), jnp.float32),
        input_output_aliases=aliases,
    )(*args)


def kernel(input_ids, word_table, pretrained_table, pos_table, gamma, beta):
    B, L = input_ids.shape
    H = word_table.shape[1]
    chunks = (768, 256)
    ids_flat = input_ids.reshape(-1).astype(jnp.int32)
    pre = pretrained_table[:L]
    pos = pos_table[:L]
    gm = gamma.reshape(1, H)
    bt = beta.reshape(1, H)

    offs = [0]
    for c in chunks:
        offs.append(offs[-1] + c)
    gathered = [
        _sc_gather(ids_flat[offs[k] * L:offs[k + 1] * L], word_table)
        for k in range(len(chunks))
    ]
    acc = None
    for k, BK in enumerate(chunks):
        acc = _tc_add_layernorm(
            gathered[k].reshape(BK, L, H), pre, pos, gm, bt, acc, offs[k], B
        )
    return acc


# speedup vs baseline: 1.0794x; 1.0414x over previous
import functools

import jax
import jax.numpy as jnp
from jax import lax
from jax.experimental import pallas as pl
from jax.experimental.pallas import tpu as pltpu
from jax.experimental.pallas import tpu_sc as plsc

_EPS = 1e-12


def _sc_gather(ids_flat, word_table):
    N = ids_flat.shape[0]
    H = word_table.shape[1]
    info = plsc.get_sparse_core_info()
    NC, NS = info.num_cores, info.num_subcores
    NW = NC * NS
    per_w = N // NW
    CH = next(c for c in range(128, 7, -8) if per_w % c == 0)
    n_ch = per_w // CH
    NB = next(b for b in (5, 4, 3, 2, 1) if n_ch % b == 0)

    mesh = plsc.VectorSubcoreMesh(core_axis_name="c", subcore_axis_name="s")

    @functools.partial(
        pl.kernel,
        mesh=mesh,
        out_type=jax.ShapeDtypeStruct((N, H), jnp.float32),
        scratch_types=[
            pltpu.VMEM((NB, CH), jnp.int32),
            pltpu.VMEM((NB, CH, H), jnp.float32),
        ]
        + [pltpu.SemaphoreType.DMA] * (2 * NB),
    )
    def k(ids_hbm, table_hbm, out_hbm, idx_v, rows_v, *sems):
        wid = lax.axis_index("s") * NC + lax.axis_index("c")
        base = wid * per_w
        gsem = sems[:NB]
        wsem = sems[NB:]

        for b in range(NB):
            off = base + b * CH
            pltpu.sync_copy(ids_hbm.at[pl.ds(off, CH)], idx_v.at[b])
            pltpu.async_copy(table_hbm.at[idx_v.at[b]], rows_v.at[b], gsem[b])

        def body(i, carry):
            for b in range(NB):
                g = i * NB + b
                off = base + g * CH
                pltpu.make_async_copy(
                    table_hbm.at[idx_v.at[b]], rows_v.at[b], gsem[b]
                ).wait()
                pltpu.async_copy(
                    rows_v.at[b], out_hbm.at[pl.ds(off, CH)], wsem[b]
                )

                @pl.when(i < n_ch // NB - 1)
                def _prefetch():
                    noff = off + NB * CH
                    pltpu.sync_copy(ids_hbm.at[pl.ds(noff, CH)], idx_v.at[b])
                    pltpu.make_async_copy(
                        rows_v.at[b], out_hbm.at[pl.ds(off, CH)], wsem[b]
                    ).wait()
                    pltpu.async_copy(
                        table_hbm.at[idx_v.at[b]], rows_v.at[b], gsem[b]
                    )

            return carry

        lax.fori_loop(0, n_ch // NB, body, 0)
        for b in range(NB):
            off = base + (n_ch - NB + b) * CH
            pltpu.make_async_copy(
                rows_v.at[b], out_hbm.at[pl.ds(off, CH)], wsem[b]
            ).wait()

    return k(ids_flat, word_table)


def _tc_add_layernorm(gathered, pre_tab, pos_tab, gamma, beta, acc, boff, full_b):
    BK, L, H = gathered.shape
    RB = 64
    grid = (BK // RB,)
    blk = boff // RB

    def body(g_ref, pa_ref, pb_ref, gm_ref, bt_ref, *rest):
        o_ref = rest[-1]
        x = g_ref[...] + (pa_ref[...] + pb_ref[...])[None, :, :]
        u = jnp.mean(x, axis=-1, keepdims=True)
        s2 = jnp.mean((x - u) ** 2, axis=-1, keepdims=True)
        xn = (x - u) * lax.rsqrt(s2 + _EPS)
        o_ref[...] = xn * gm_ref[0][None, None, :] + bt_ref[0][None, None, :]

    in_specs = [
        pl.BlockSpec((RB, L, H), lambda i: (i, 0, 0)),
        pl.BlockSpec((L, H), lambda i: (0, 0)),
        pl.BlockSpec((L, H), lambda i: (0, 0)),
        pl.BlockSpec((1, H), lambda i: (0, 0)),
        pl.BlockSpec((1, H), lambda i: (0, 0)),
    ]
    args = [gathered, pre_tab, pos_tab, gamma, beta]
    aliases = {}
    if acc is not None:
        in_specs.append(pl.BlockSpec((RB, L, H), lambda i: (i + blk, 0, 0)))
        args.append(acc)
        aliases = {5: 0}

    return pl.pallas_call(
        body,
        grid=grid,
        in_specs=in_specs,
        out_specs=pl.BlockSpec((RB, L, H), lambda i: (i + blk, 0, 0)),
        out_shape=jax.ShapeDtypeStruct((full_b, L, H), jnp.float32),
        input_output_aliases=aliases,
    )(*args)


def kernel(input_ids, word_table, pretrained_table, pos_table, gamma, beta):
    B, L = input_ids.shape
    H = word_table.shape[1]
    chunks = (1024,)
    ids_flat = input_ids.reshape(-1).astype(jnp.int32)
    pre = pretrained_table[:L]
    pos = pos_table[:L]
    gm = gamma.reshape(1, H)
    bt = beta.reshape(1, H)

    offs = [0]
    for c in chunks:
        offs.append(offs[-1] + c)
    gathered = [
        _sc_gather(ids_flat[offs[k] * L:offs[k + 1] * L], word_table)
        for k in range(len(chunks))
    ]
    acc = None
    for k, BK in enumerate(chunks):
        acc = _tc_add_layernorm(
            gathered[k].reshape(BK, L, H), pre, pos, gm, bt, acc, offs[k], B
        )
    return acc
